# Initial kernel scaffold; baseline (speedup 1.0000x reference)
import functools

import jax
import jax.numpy as jnp
import numpy as np
from jax import lax
from jax.experimental import pallas as pl
from jax.experimental.pallas import tpu as pltpu
from jax.experimental.pallas import tpu_sc as plsc

N = 10000
E = 320000
D = 128
EPS = 1e-5

NC = 2
NS = 16
NW = NC * NS
CHUNK = 128
K = 79
EPT = K * CHUNK
EPAD = NW * EPT
NPAD = 10240
RPT = NPAD // NS

_mesh = plsc.VectorSubcoreMesh(core_axis_name="c", subcore_axis_name="s")


@functools.partial(
    pl.kernel,
    out_type=(
        jax.ShapeDtypeStruct((NC, NPAD, D), jnp.float32),
        jax.ShapeDtypeStruct((NC, NPAD), jnp.float32),
    ),
    mesh=_mesh,
    scratch_types=[
        pltpu.VMEM((K, CHUNK), jnp.int32),
        pltpu.VMEM((K, CHUNK), jnp.int32),
        pltpu.VMEM((K, CHUNK), jnp.float32),
        pltpu.VMEM((CHUNK, D), jnp.float32),
        pltpu.VMEM_SHARED((NPAD, D), jnp.float32),
        pltpu.VMEM_SHARED((NPAD,), jnp.float32),
        pltpu.SemaphoreType.DMA,
    ],
)
def _sc_scatter(nr_hbm, src_hbm, dst_hbm, adv_hbm, zf_hbm, zd_hbm,
                out_s, out_d,
                src_v, dst_v, adv_v, rows_v, accum, denom, sem):
    c = lax.axis_index("c")
    s = lax.axis_index("s")
    wid = c * NS + s

    pltpu.sync_copy(zf_hbm.at[pl.ds(s * RPT, RPT)], accum.at[pl.ds(s * RPT, RPT)])
    pltpu.sync_copy(zd_hbm.at[pl.ds(s * RPT, RPT)], denom.at[pl.ds(s * RPT, RPT)])

    pltpu.sync_copy(src_hbm.at[wid], src_v)
    pltpu.sync_copy(dst_hbm.at[wid], dst_v)
    pltpu.sync_copy(adv_hbm.at[wid], adv_v)

    plsc.subcore_barrier()

    def chunk_body(k, carry):
        pltpu.async_copy(nr_hbm.at[src_v.at[k]], rows_v, sem).wait()

        def edge_body(i, carry2):
            a = adv_v[k, i]
            av = jnp.full((16,), a, jnp.float32)
            for j in range(8):
                sl = pl.ds(j * 16, 16)
                rows_v[i, sl] = rows_v[i, sl] * av
            return carry2

        lax.fori_loop(0, CHUNK, edge_body, 0, unroll=False)

        pltpu.sync_copy(rows_v, accum.at[dst_v.at[k]], add=True)
        pltpu.sync_copy(adv_v.at[k], denom.at[dst_v.at[k]], add=True)
        return carry

    lax.fori_loop(0, K, chunk_body, 0, unroll=False)

    plsc.subcore_barrier()

    pltpu.sync_copy(accum.at[pl.ds(s * RPT, RPT)], out_s.at[c, pl.ds(s * RPT, RPT)])
    pltpu.sync_copy(denom.at[pl.ds(s * RPT, RPT)], out_d.at[c, pl.ds(s * RPT, RPT)])


_RB = 1024


def _proj_body(x_ref, w_ref, b_ref, o_ref):
    o_ref[...] = (
        jnp.dot(x_ref[...], w_ref[...], preferred_element_type=jnp.float32)
        + b_ref[...]
    )


def _proj(x, w, b):
    return pl.pallas_call(
        _proj_body,
        grid=(NPAD // _RB,),
        in_specs=[
            pl.BlockSpec((_RB, D), lambda i: (i, 0)),
            pl.BlockSpec((D, D), lambda i: (0, 0)),
            pl.BlockSpec((1, D), lambda i: (0, 0)),
        ],
        out_specs=pl.BlockSpec((_RB, D), lambda i: (i, 0)),
        out_shape=jax.ShapeDtypeStruct((NPAD, D), jnp.float32),
    )(x, w, b.reshape(1, D))


_SQRT_HALF = np.float32(1.0 / np.sqrt(2.0))


def _combine_body(sp_ref, dp_ref, nr_ref, w_ref, b_ref, g_ref, be_ref, o_ref):
    ssum = sp_ref[0] + sp_ref[1]
    den = dp_ref[0] + dp_ref[1]
    aggr = jnp.where(den > 0.0, ssum / den, 0.0)
    h = (
        jnp.dot(aggr + nr_ref[...], w_ref[...], preferred_element_type=jnp.float32)
        + b_ref[...]
    )
    h = 0.5 * h * (1.0 + lax.erf(h * _SQRT_HALF))
    mu = jnp.mean(h, axis=-1, keepdims=True)
    var = jnp.mean((h - mu) ** 2, axis=-1, keepdims=True)
    o_ref[...] = (h - mu) / jnp.sqrt(var + EPS) * g_ref[...] + be_ref[...]


def _combine(sp, dp, nr, w, b, g, be):
    return pl.pallas_call(
        _combine_body,
        grid=(NPAD // _RB,),
        in_specs=[
            pl.BlockSpec((NC, _RB, D), lambda i: (0, i, 0)),
            pl.BlockSpec((NC, _RB, 1), lambda i: (0, i, 0)),
            pl.BlockSpec((_RB, D), lambda i: (i, 0)),
            pl.BlockSpec((D, D), lambda i: (0, 0)),
            pl.BlockSpec((1, D), lambda i: (0, 0)),
            pl.BlockSpec((1, D), lambda i: (0, 0)),
            pl.BlockSpec((1, D), lambda i: (0, 0)),
        ],
        out_specs=pl.BlockSpec((_RB, D), lambda i: (i, 0)),
        out_shape=jax.ShapeDtypeStruct((NPAD, D), jnp.float32),
    )(sp, dp.reshape(NC, NPAD, 1), nr, w, b.reshape(1, D), g.reshape(1, D),
      be.reshape(1, D))


def kernel(node_attr, edge_index, batch_idx, adv_atts,
           W_in, b_in, W_l0, b_l0, g_l0, be_l0,
           W_l1, b_l1, g_l1, be_l1, W_out, b_out):
    src = edge_index[0]
    dst = edge_index[1]

    pad_e = EPAD - E
    src_p = jnp.concatenate([src, jnp.full((pad_e,), N, jnp.int32)]).reshape(NW, K, CHUNK)
    dst_p = jnp.concatenate([dst, jnp.full((pad_e,), N, jnp.int32)]).reshape(NW, K, CHUNK)
    adv_p = jnp.concatenate(
        [adv_atts, jnp.zeros((2, pad_e), jnp.float32)], axis=1
    ).reshape(2, NW, K, CHUNK)

    x = jnp.zeros((NPAD, D), jnp.float32).at[:N].set(node_attr)
    zf = jnp.zeros((NPAD, D), jnp.float32)
    zd = jnp.zeros((NPAD,), jnp.float32)

    nr = _proj(x, W_in, b_in)

    sp, dp = _sc_scatter(nr, src_p, dst_p, adv_p[0], zf, zd)
    nr = _combine(sp, dp, nr, W_l0, b_l0, g_l0, be_l0)

    sp, dp = _sc_scatter(nr, src_p, dst_p, adv_p[1], zf, zd)
    nr = _combine(sp, dp, nr, W_l1, b_l1, g_l1, be_l1)

    out = _proj(nr, W_out, b_out)
    return out[:N]


# speedup vs baseline: 12.3440x; 12.3440x over previous
import functools

import jax
import jax.numpy as jnp
import numpy as np
from jax import lax
from jax.experimental import pallas as pl
from jax.experimental.pallas import tpu as pltpu
from jax.experimental.pallas import tpu_sc as plsc

N = 10000
E = 320000
D = 128
EPS = 1e-5

NC = 2
NS = 16
NW = NC * NS
CHUNK = 128
K = 79
EPT = K * CHUNK
EPAD = NW * EPT
NPAD = 10240
RPT = NPAD // NS

_mesh = plsc.VectorSubcoreMesh(core_axis_name="c", subcore_axis_name="s")


@functools.partial(
    pl.kernel,
    out_type=(
        jax.ShapeDtypeStruct((NC, NPAD, D), jnp.float32),
        jax.ShapeDtypeStruct((NC, NPAD), jnp.float32),
    ),
    mesh=_mesh,
    scratch_types=[
        pltpu.VMEM((K, CHUNK), jnp.int32),
        pltpu.VMEM((K, CHUNK), jnp.int32),
        pltpu.VMEM((K, CHUNK), jnp.float32),
        pltpu.VMEM((CHUNK, D), jnp.float32),
        pltpu.VMEM_SHARED((NPAD, D), jnp.float32),
        pltpu.VMEM_SHARED((NPAD,), jnp.float32),
        pltpu.SemaphoreType.DMA,
    ],
)
def _sc_scatter(nr_hbm, src_hbm, dst_hbm, adv_hbm, zf_hbm, zd_hbm,
                out_s, out_d,
                src_v, dst_v, adv_v, rows_v, accum, denom, sem):
    c = lax.axis_index("c")
    s = lax.axis_index("s")
    wid = c * NS + s

    pltpu.sync_copy(zf_hbm.at[pl.ds(s * RPT, RPT)], accum.at[pl.ds(s * RPT, RPT)])
    pltpu.sync_copy(zd_hbm.at[pl.ds(s * RPT, RPT)], denom.at[pl.ds(s * RPT, RPT)])

    pltpu.sync_copy(src_hbm.at[wid], src_v)
    pltpu.sync_copy(dst_hbm.at[wid], dst_v)
    pltpu.sync_copy(adv_hbm.at[wid], adv_v)

    plsc.subcore_barrier()

    def chunk_body(k, carry):
        pltpu.async_copy(nr_hbm.at[src_v.at[k]], rows_v, sem).wait()

        def group_body(g, carry2):
            base = g * 16
            av16 = adv_v[k, pl.ds(base, 16)]
            for t in range(16):
                av = jnp.full((16,), av16[t], jnp.float32)
                i = base + t
                for j in range(8):
                    sl = pl.ds(j * 16, 16)
                    rows_v[i, sl] = rows_v[i, sl] * av
            return carry2

        lax.fori_loop(0, CHUNK // 16, group_body, 0, unroll=False)

        pltpu.sync_copy(rows_v, accum.at[dst_v.at[k]], add=True)
        pltpu.sync_copy(adv_v.at[k], denom.at[dst_v.at[k]], add=True)
        return carry

    lax.fori_loop(0, K, chunk_body, 0, unroll=False)

    plsc.subcore_barrier()

    pltpu.sync_copy(accum.at[pl.ds(s * RPT, RPT)], out_s.at[c, pl.ds(s * RPT, RPT)])
    pltpu.sync_copy(denom.at[pl.ds(s * RPT, RPT)], out_d.at[c, pl.ds(s * RPT, RPT)])


_RB = 1024


def _proj_body(x_ref, w_ref, b_ref, o_ref):
    o_ref[...] = (
        jnp.dot(x_ref[...], w_ref[...], preferred_element_type=jnp.float32)
        + b_ref[...]
    )


def _proj(x, w, b):
    return pl.pallas_call(
        _proj_body,
        grid=(NPAD // _RB,),
        in_specs=[
            pl.BlockSpec((_RB, D), lambda i: (i, 0)),
            pl.BlockSpec((D, D), lambda i: (0, 0)),
            pl.BlockSpec((1, D), lambda i: (0, 0)),
        ],
        out_specs=pl.BlockSpec((_RB, D), lambda i: (i, 0)),
        out_shape=jax.ShapeDtypeStruct((NPAD, D), jnp.float32),
    )(x, w, b.reshape(1, D))


_SQRT_HALF = np.float32(1.0 / np.sqrt(2.0))


def _combine_body(sp_ref, dp_ref, nr_ref, w_ref, b_ref, g_ref, be_ref, o_ref):
    ssum = sp_ref[0] + sp_ref[1]
    den = dp_ref[0] + dp_ref[1]
    aggr = jnp.where(den > 0.0, ssum / den, 0.0)
    h = (
        jnp.dot(aggr + nr_ref[...], w_ref[...], preferred_element_type=jnp.float32)
        + b_ref[...]
    )
    h = 0.5 * h * (1.0 + lax.erf(h * _SQRT_HALF))
    mu = jnp.mean(h, axis=-1, keepdims=True)
    var = jnp.mean((h - mu) ** 2, axis=-1, keepdims=True)
    o_ref[...] = (h - mu) / jnp.sqrt(var + EPS) * g_ref[...] + be_ref[...]


def _combine(sp, dp, nr, w, b, g, be):
    return pl.pallas_call(
        _combine_body,
        grid=(NPAD // _RB,),
        in_specs=[
            pl.BlockSpec((NC, _RB, D), lambda i: (0, i, 0)),
            pl.BlockSpec((NC, _RB, 1), lambda i: (0, i, 0)),
            pl.BlockSpec((_RB, D), lambda i: (i, 0)),
            pl.BlockSpec((D, D), lambda i: (0, 0)),
            pl.BlockSpec((1, D), lambda i: (0, 0)),
            pl.BlockSpec((1, D), lambda i: (0, 0)),
            pl.BlockSpec((1, D), lambda i: (0, 0)),
        ],
        out_specs=pl.BlockSpec((_RB, D), lambda i: (i, 0)),
        out_shape=jax.ShapeDtypeStruct((NPAD, D), jnp.float32),
    )(sp, dp.reshape(NC, NPAD, 1), nr, w, b.reshape(1, D), g.reshape(1, D),
      be.reshape(1, D))


def kernel(node_attr, edge_index, batch_idx, adv_atts,
           W_in, b_in, W_l0, b_l0, g_l0, be_l0,
           W_l1, b_l1, g_l1, be_l1, W_out, b_out):
    src = edge_index[0]
    dst = edge_index[1]

    pad_e = EPAD - E
    src_p = jnp.concatenate([src, jnp.full((pad_e,), N, jnp.int32)]).reshape(NW, K, CHUNK)
    dst_p = jnp.concatenate([dst, jnp.full((pad_e,), N, jnp.int32)]).reshape(NW, K, CHUNK)
    adv_p = jnp.concatenate(
        [adv_atts, jnp.zeros((2, pad_e), jnp.float32)], axis=1
    ).reshape(2, NW, K, CHUNK)

    x = jnp.zeros((NPAD, D), jnp.float32).at[:N].set(node_attr)
    zf = jnp.zeros((NPAD, D), jnp.float32)
    zd = jnp.zeros((NPAD,), jnp.float32)

    nr = _proj(x, W_in, b_in)

    sp, dp = _sc_scatter(nr, src_p, dst_p, adv_p[0], zf, zd)
    nr = _combine(sp, dp, nr, W_l0, b_l0, g_l0, be_l0)

    sp, dp = _sc_scatter(nr, src_p, dst_p, adv_p[1], zf, zd)
    nr = _combine(sp, dp, nr, W_l1, b_l1, g_l1, be_l1)

    out = _proj(nr, W_out, b_out)
    return out[:N]
